# baseline (speedup 1.0000x reference)
import functools

import jax
import jax.numpy as jnp
from jax import lax
from jax.experimental import pallas as pl
from jax.experimental.pallas import tpu as pltpu
from jax.experimental.pallas import tpu_sc as plsc

_NC = 2
_NS = 16
_L = 16
_NW = _NC * _NS


def kernel(target, vector_embedding, centers):
    B, D = vector_embedding.shape
    V = centers.shape[0]
    n_pass = D // _NW
    n_q = 4
    BQ = B // n_q
    VH0 = ((V // 2 + 127) // 128) * 128
    VH1 = V - VH0
    TAIL = VH1 % 128
    VH1M = VH1 - TAIL
    n_task = n_pass * 2

    cen_t = centers.T
    emb_t = vector_embedding.T
    cen_tail = jnp.pad(cen_t[:, V - TAIL:], ((0, 0), (0, 128 - TAIL)))

    mesh = plsc.VectorSubcoreMesh(core_axis_name="c", subcore_axis_name="s")

    @functools.partial(
        pl.kernel,
        mesh=mesh,
        out_type=jax.ShapeDtypeStruct((_NW, _L), jnp.float32),
        scratch_types=[
            pltpu.VMEM((VH0,), jnp.float32),
            pltpu.VMEM((VH0,), jnp.float32),
            pltpu.VMEM((2, BQ), jnp.float32),
            pltpu.VMEM((B,), jnp.int32),
            pltpu.VMEM((_L,), jnp.float32),
            pltpu.SemaphoreType.DMA,
            pltpu.SemaphoreType.DMA,
            pltpu.SemaphoreType.DMA,
            pltpu.SemaphoreType.DMA,
            pltpu.SemaphoreType.DMA,
            pltpu.SemaphoreType.DMA,
        ],
        compiler_params=pltpu.CompilerParams(needs_layout_passes=False),
    )
    def sc_kernel(tgt_hbm, emb_hbm, cen_hbm, tail_hbm, out_hbm, row_v0, row_v1,
                  emb_v, idx_v, acc_v, sem_i, sem_r0, sem_r1, sem_e0, sem_e1,
                  sem_o):
        wid = lax.axis_index("s") * _NC + lax.axis_index("c")
        rows = (row_v0, row_v1)
        rsems = (sem_r0, sem_r1)
        esems = (sem_e0, sem_e1)
        halves = ((0, VH0), (VH0, VH1))

        def row_cp(t):
            p, c = t // 2, t % 2
            f = wid + p * _NW
            if c == 0:
                return [pltpu.async_copy(cen_hbm.at[f, pl.ds(0, VH0)],
                                         rows[t % 2].at[pl.ds(0, VH0)],
                                         rsems[t % 2])]
            return [
                pltpu.async_copy(cen_hbm.at[f, pl.ds(VH0, VH1M)],
                                 rows[t % 2].at[pl.ds(0, VH1M)],
                                 rsems[t % 2]),
                pltpu.async_copy(tail_hbm.at[f],
                                 rows[t % 2].at[pl.ds(VH1M, 128)],
                                 rsems[t % 2]),
            ]

        def emb_cp(gq):
            t, q = gq // n_q, gq % n_q
            return pltpu.async_copy(
                emb_hbm.at[wid + (t // 2) * _NW, pl.ds(q * BQ, BQ)],
                emb_v.at[gq % 2], esems[gq % 2])

        cp_i = pltpu.async_copy(tgt_hbm, idx_v, sem_i)
        rows_pending = [row_cp(0), row_cp(1)]
        emb_pending = [emb_cp(0), emb_cp(1)]
        cp_i.wait()

        zero = jnp.zeros((_L,), jnp.float32)
        accs = (zero, zero)
        for t in range(n_task):
            for _cp in rows_pending[t % 2]:
                _cp.wait()
            rref = rows[t % 2]
            cbase, vh = halves[t % 2]
            for q in range(n_q):
                gq = t * n_q + q
                emb_pending[gq % 2].wait()
                ebuf = gq % 2
                ib = q * BQ

                @plsc.parallel_loop(0, BQ // (2 * _L), unroll=4, carry=accs)
                def body(i, accs):
                    a0, a1 = accs
                    b = i * (2 * _L)

                    t0 = idx_v[pl.ds(ib + b, _L)] - cbase
                    l0 = jnp.minimum(jnp.maximum(t0, 0), vh - 1)
                    e0 = emb_v[ebuf, pl.ds(b, _L)]
                    g0 = plsc.load_gather(rref, [l0])
                    d0 = jnp.where(t0 == l0, e0 - g0, 0.0)

                    t1 = idx_v[pl.ds(ib + b + _L, _L)] - cbase
                    l1 = jnp.minimum(jnp.maximum(t1, 0), vh - 1)
                    e1 = emb_v[ebuf, pl.ds(b + _L, _L)]
                    g1 = plsc.load_gather(rref, [l1])
                    d1 = jnp.where(t1 == l1, e1 - g1, 0.0)

                    return (a0 + d0 * d0, a1 + d1 * d1)

                accs = body
                ngq = gq + 2
                if ngq < n_task * n_q:
                    emb_pending[gq % 2] = emb_cp(ngq)
            if t + 2 < n_task:
                rows_pending[t % 2] = row_cp(t + 2)

        acc_v[...] = (accs[0] + accs[1]) * (0.5 / B)
        pltpu.sync_copy(acc_v, out_hbm.at[wid])

    partials = sc_kernel(target, emb_t, cen_t, cen_tail)
    return jnp.sum(partials)
